# baseline (speedup 1.0000x reference)
import jax
import jax.numpy as jnp
from jax.experimental import pallas as pl
from jax.experimental.pallas import tpu as pltpu

B = 2
L = 2048
D = 768
K = 8
LP = L + K


def _tc_body(xp_ref, out_ref):
    k = pl.program_id(1)
    a = xp_ref[0]
    e = pltpu.roll(a, (LP - k) % LP, axis=0)
    out_ref[0, :, :] = (a[0:L, :] + e[0:L, :]) * 0.5


@jax.jit
def kernel(x):
    xp = jnp.pad(x, ((0, 0), (0, K), (0, 0)))
    out = pl.pallas_call(
        _tc_body,
        grid=(B, K),
        in_specs=[pl.BlockSpec((1, LP, D), lambda b, k: (b, 0, 0))],
        out_specs=pl.BlockSpec((1, L, D), lambda b, k: (b, 0, k)),
        out_shape=jax.ShapeDtypeStruct((B, L, K * D), jnp.float32),
    )(xp)
    return out.reshape(B, L, K, D)


# speedup vs baseline: 1.7629x; 1.7629x over previous
import jax
import jax.numpy as jnp
from jax import lax
from jax.experimental import pallas as pl
from jax.experimental.pallas import tpu as pltpu
from jax.experimental.pallas import tpu_sc as plsc

B = 2
L = 2048
D = 768
K = 8

NC = 2
NS = 16
NW = NC * NS
P = B * L
PPW = P // NW
R = 8
C = PPW // R
HALO = 8
LANES = 16
NJ = D // LANES


def _body(x_hbm, out_hbm, in_a, in_b, out_a, out_b, si_a, si_b, so_a, so_b):
    wid = lax.axis_index("s") * NC + lax.axis_index("c")
    base = wid * PPW

    def in_copies(c, buf, sem):
        p0 = base + c * R
        h0 = jnp.minimum(p0 + R, P - HALO)
        d1 = pltpu.make_async_copy(x_hbm.at[pl.ds(p0, R)], buf.at[pl.ds(0, R)], sem)
        d2 = pltpu.make_async_copy(x_hbm.at[pl.ds(h0, HALO)], buf.at[pl.ds(R, HALO)], sem)
        return d1, d2

    def start_in(c, buf, sem):
        for d in in_copies(c, buf, sem):
            d.start()

    def wait_in(c, buf, sem):
        for d in in_copies(c, buf, sem):
            d.wait()

    def out_copy(c, buf, sem):
        p0 = base + c * R
        return pltpu.make_async_copy(buf, out_hbm.at[pl.ds(p0 * K, R * K)], sem)

    def compute_fast(ib, ob):
        @plsc.parallel_loop(0, NJ, 1, unroll=4)
        def body_j(j):
            col = pl.ds(j * LANES, LANES)
            for r in range(R):
                a = ib[r, col]
                ob[r * K, col] = a
                for k in range(1, K):
                    b = ib[r + k, col]
                    ob[r * K + k, col] = (a + b) * 0.5

    def compute_edge(ib, ob, p0, batch_end):
        @plsc.parallel_loop(0, NJ, 1, unroll=2)
        def body_j(j):
            col = pl.ds(j * LANES, LANES)
            for r in range(R):
                a = ib[r, col]
                ob[r * K, col] = a
                for k in range(1, K):
                    b = ib[r + k, col]
                    m = jnp.where(p0 + r + k < batch_end,
                                  jnp.float32(0.5), jnp.float32(0.0))
                    ob[r * K + k, col] = a * 0.5 + b * m

    start_in(0, in_a, si_a)
    start_in(1, in_b, si_b)

    wait_in(0, in_a, si_a)
    compute_fast(in_a, out_a)
    out_copy(0, out_a, so_a).start()
    start_in(2, in_a, si_a)

    wait_in(1, in_b, si_b)
    compute_fast(in_b, out_b)
    out_copy(1, out_b, so_b).start()
    start_in(3, in_b, si_b)

    def pair(cc, carry):
        c0 = cc * 2
        wait_in(c0, in_a, si_a)
        out_copy(c0 - 2, out_a, so_a).wait()
        compute_fast(in_a, out_a)
        out_copy(c0, out_a, so_a).start()
        start_in(c0 + 2, in_a, si_a)

        c1 = c0 + 1
        wait_in(c1, in_b, si_b)
        out_copy(c1 - 2, out_b, so_b).wait()
        compute_fast(in_b, out_b)
        out_copy(c1, out_b, so_b).start()
        start_in(c1 + 2, in_b, si_b)
        return carry

    lax.fori_loop(1, C // 2 - 1, pair, 0)

    wait_in(C - 2, in_a, si_a)
    out_copy(C - 4, out_a, so_a).wait()
    compute_fast(in_a, out_a)
    out_copy(C - 2, out_a, so_a).start()

    p0_t = base + (C - 1) * R
    batch_end = (p0_t // L + 1) * L
    is_edge = (p0_t + R + K - 2) >= batch_end
    wait_in(C - 1, in_b, si_b)
    out_copy(C - 3, out_b, so_b).wait()
    pl.when(jnp.logical_not(is_edge))(lambda: compute_fast(in_b, out_b))
    pl.when(is_edge)(lambda: compute_edge(in_b, out_b, p0_t, batch_end))
    out_copy(C - 1, out_b, so_b).start()

    out_copy(C - 2, out_a, so_a).wait()
    out_copy(C - 1, out_b, so_b).wait()


@jax.jit
def kernel(x):
    xf = x.reshape(P, D)
    mesh = plsc.VectorSubcoreMesh(core_axis_name="c", subcore_axis_name="s")
    run = pl.kernel(
        _body,
        out_type=jax.ShapeDtypeStruct((P * K, D), jnp.float32),
        mesh=mesh,
        scratch_types=[
            pltpu.VMEM((R + HALO, D), jnp.float32),
            pltpu.VMEM((R + HALO, D), jnp.float32),
            pltpu.VMEM((R * K, D), jnp.float32),
            pltpu.VMEM((R * K, D), jnp.float32),
            pltpu.SemaphoreType.DMA,
            pltpu.SemaphoreType.DMA,
            pltpu.SemaphoreType.DMA,
            pltpu.SemaphoreType.DMA,
        ],
    )
    out = run(xf)
    return out.reshape(B, L, K, D)


# speedup vs baseline: 2.3324x; 1.3230x over previous
import jax
import jax.numpy as jnp
from jax import lax
from jax.experimental import pallas as pl
from jax.experimental.pallas import tpu as pltpu
from jax.experimental.pallas import tpu_sc as plsc

B = 2
L = 2048
D = 768
K = 8

NC = 2
NS = 16
NW = NC * NS
P = B * L
PPW = P // NW
R = 8
C = PPW // R
HALO = 8
LANES = 16
NJ = D // LANES


def _body(x_hbm, out_hbm, in_a, in_b, out_a, out_b, si_a, si_b, so_a, so_b):
    wid = lax.axis_index("s") * NC + lax.axis_index("c")
    base = wid * PPW

    def in_copies(c, buf, sem):
        p0 = base + c * R
        h0 = jnp.minimum(p0 + R, P - HALO)
        d1 = pltpu.make_async_copy(x_hbm.at[pl.ds(p0, R)], buf.at[pl.ds(0, R)], sem)
        d2 = pltpu.make_async_copy(x_hbm.at[pl.ds(h0, HALO)], buf.at[pl.ds(R, HALO)], sem)
        return d1, d2

    def start_in(c, buf, sem):
        for d in in_copies(c, buf, sem):
            d.start()

    def wait_in(c, buf, sem):
        for d in in_copies(c, buf, sem):
            d.wait()

    def out_copy(c, buf, sem):
        p0 = base + c * R
        return pltpu.make_async_copy(buf, out_hbm.at[pl.ds(p0 * K, R * K)], sem)

    def compute_fast(ib, ob):
        def body_j(j, cc):
            col = pl.ds(j * LANES, LANES)
            for r in range(R):
                a = ib[r, col]
                ob[r * K, col] = a
                for k in range(1, K):
                    b = ib[r + k, col]
                    ob[r * K + k, col] = (a + b) * 0.5
            return cc
        lax.fori_loop(0, NJ, body_j, 0)

    def compute_edge(ib, ob, p0, batch_end):
        def body_j(j, cc):
            col = pl.ds(j * LANES, LANES)
            for r in range(R):
                a = ib[r, col]
                ob[r * K, col] = a
                for k in range(1, K):
                    b = ib[r + k, col]
                    m = jnp.where(p0 + r + k < batch_end,
                                  jnp.float32(0.5), jnp.float32(0.0))
                    ob[r * K + k, col] = a * 0.5 + b * m
            return cc
        lax.fori_loop(0, NJ, body_j, 0)

    start_in(0, in_a, si_a)
    start_in(1, in_b, si_b)

    wait_in(0, in_a, si_a)
    compute_fast(in_a, out_a)
    out_copy(0, out_a, so_a).start()
    start_in(2, in_a, si_a)

    wait_in(1, in_b, si_b)
    compute_fast(in_b, out_b)
    out_copy(1, out_b, so_b).start()
    start_in(3, in_b, si_b)

    def pair(cc, carry):
        c0 = cc * 2
        wait_in(c0, in_a, si_a)
        out_copy(c0 - 2, out_a, so_a).wait()
        compute_fast(in_a, out_a)
        out_copy(c0, out_a, so_a).start()
        start_in(c0 + 2, in_a, si_a)

        c1 = c0 + 1
        wait_in(c1, in_b, si_b)
        out_copy(c1 - 2, out_b, so_b).wait()
        compute_fast(in_b, out_b)
        out_copy(c1, out_b, so_b).start()
        start_in(c1 + 2, in_b, si_b)
        return carry

    lax.fori_loop(1, C // 2 - 1, pair, 0)

    wait_in(C - 2, in_a, si_a)
    out_copy(C - 4, out_a, so_a).wait()
    compute_fast(in_a, out_a)
    out_copy(C - 2, out_a, so_a).start()

    p0_t = base + (C - 1) * R
    batch_end = (p0_t // L + 1) * L
    is_edge = (p0_t + R + K - 2) >= batch_end
    wait_in(C - 1, in_b, si_b)
    out_copy(C - 3, out_b, so_b).wait()
    pl.when(jnp.logical_not(is_edge))(lambda: compute_fast(in_b, out_b))
    pl.when(is_edge)(lambda: compute_edge(in_b, out_b, p0_t, batch_end))
    out_copy(C - 1, out_b, so_b).start()

    out_copy(C - 2, out_a, so_a).wait()
    out_copy(C - 1, out_b, so_b).wait()


@jax.jit
def kernel(x):
    xf = x.reshape(P, D)
    mesh = plsc.VectorSubcoreMesh(core_axis_name="c", subcore_axis_name="s")
    run = pl.kernel(
        _body,
        out_type=jax.ShapeDtypeStruct((P * K, D), jnp.float32),
        mesh=mesh,
        scratch_types=[
            pltpu.VMEM((R + HALO, D), jnp.float32),
            pltpu.VMEM((R + HALO, D), jnp.float32),
            pltpu.VMEM((R * K, D), jnp.float32),
            pltpu.VMEM((R * K, D), jnp.float32),
            pltpu.SemaphoreType.DMA,
            pltpu.SemaphoreType.DMA,
            pltpu.SemaphoreType.DMA,
            pltpu.SemaphoreType.DMA,
        ],
    )
    out = run(xf)
    return out.reshape(B, L, K, D)


# speedup vs baseline: 2.3967x; 1.0276x over previous
import jax
import jax.numpy as jnp
from jax import lax
from jax.experimental import pallas as pl
from jax.experimental.pallas import tpu as pltpu
from jax.experimental.pallas import tpu_sc as plsc

B = 2
L = 2048
D = 768
K = 8

NC = 2
NS = 16
NW = NC * NS
P = B * L
PPW = P // NW
R = 8
C = PPW // R
HALO = 8
LANES = 16
NJ = D // LANES


def _body(x_hbm, out_hbm, in_a, in_b, out_a, out_b, si_a, si_b, so_a, so_b):
    wid = lax.axis_index("s") * NC + lax.axis_index("c")
    base = wid * PPW

    def in_copies(c, buf, sem):
        p0 = base + c * R
        h0 = jnp.minimum(p0 + R, P - HALO)
        d1 = pltpu.make_async_copy(x_hbm.at[pl.ds(p0, R)], buf.at[pl.ds(0, R)], sem)
        d2 = pltpu.make_async_copy(x_hbm.at[pl.ds(h0, HALO)], buf.at[pl.ds(R, HALO)], sem)
        return d1, d2

    def start_in(c, buf, sem):
        for d in in_copies(c, buf, sem):
            d.start()

    def wait_in(c, buf, sem):
        for d in in_copies(c, buf, sem):
            d.wait()

    def out_copy(c, buf, sem):
        p0 = base + c * R
        return pltpu.make_async_copy(buf, out_hbm.at[pl.ds(p0 * K, R * K)], sem)

    def compute_fast(ib, ob):
        def body_j(j, cc):
            col = pl.ds(j * LANES, LANES)
            for r in range(R):
                a = ib[r, col]
                ob[r * K, col] = a
                for k in range(1, K):
                    b = ib[r + k, col]
                    ob[r * K + k, col] = (a + b) * 0.5
            return cc
        lax.fori_loop(0, NJ, body_j, 0)

    def compute_edge(ib, ob, p0, batch_end):
        def body_j(j, cc):
            col = pl.ds(j * LANES, LANES)
            for r in range(R):
                a = ib[r, col]
                ob[r * K, col] = a
                for k in range(1, K):
                    b = ib[r + k, col]
                    m = jnp.where(p0 + r + k < batch_end,
                                  jnp.float32(0.5), jnp.float32(0.0))
                    ob[r * K + k, col] = a * 0.5 + b * m
            return cc
        lax.fori_loop(0, NJ, body_j, 0)

    start_in(0, in_a, si_a)
    start_in(1, in_b, si_b)

    def pair(cc, carry):
        not_first = cc > 0
        not_last = cc < C // 2 - 1

        c0 = cc * 2
        wait_in(c0, in_a, si_a)
        pl.when(not_first)(lambda: out_copy(c0 - 2, out_a, so_a).wait())
        compute_fast(in_a, out_a)
        out_copy(c0, out_a, so_a).start()
        pl.when(not_last)(lambda: start_in(c0 + 2, in_a, si_a))

        c1 = c0 + 1
        p1 = base + c1 * R
        batch_end = (p1 // L + 1) * L
        is_edge = (p1 + R + K - 2) >= batch_end
        wait_in(c1, in_b, si_b)
        pl.when(not_first)(lambda: out_copy(c1 - 2, out_b, so_b).wait())
        pl.when(jnp.logical_not(is_edge))(lambda: compute_fast(in_b, out_b))
        pl.when(is_edge)(lambda: compute_edge(in_b, out_b, p1, batch_end))
        out_copy(c1, out_b, so_b).start()
        pl.when(not_last)(lambda: start_in(c1 + 2, in_b, si_b))
        return carry

    lax.fori_loop(0, C // 2, pair, 0)

    out_copy(C - 2, out_a, so_a).wait()
    out_copy(C - 1, out_b, so_b).wait()


@jax.jit
def kernel(x):
    xf = x.reshape(P, D)
    mesh = plsc.VectorSubcoreMesh(core_axis_name="c", subcore_axis_name="s")
    run = pl.kernel(
        _body,
        out_type=jax.ShapeDtypeStruct((P * K, D), jnp.float32),
        mesh=mesh,
        scratch_types=[
            pltpu.VMEM((R + HALO, D), jnp.float32),
            pltpu.VMEM((R + HALO, D), jnp.float32),
            pltpu.VMEM((R * K, D), jnp.float32),
            pltpu.VMEM((R * K, D), jnp.float32),
            pltpu.SemaphoreType.DMA,
            pltpu.SemaphoreType.DMA,
            pltpu.SemaphoreType.DMA,
            pltpu.SemaphoreType.DMA,
        ],
    )
    out = run(xf)
    return out.reshape(B, L, K, D)


# speedup vs baseline: 2.3984x; 1.0007x over previous
import jax
import jax.numpy as jnp
from jax import lax
from jax.experimental import pallas as pl
from jax.experimental.pallas import tpu as pltpu
from jax.experimental.pallas import tpu_sc as plsc

B = 2
L = 2048
D = 768
K = 8

NC = 2
NS = 16
NW = NC * NS
P = B * L
PPW = P // NW
R = 8
C = PPW // R
HALO = 8
LANES = 16
NJ = D // LANES


def _body(x_hbm, out_hbm, in_a, in_b, out_a, out_b, si_a, si_b, so_a, so_b):
    wid = lax.axis_index("c") * NS + lax.axis_index("s")
    base = wid * PPW

    def in_copies(c, buf, sem):
        p0 = base + c * R
        h0 = jnp.minimum(p0 + R, P - HALO)
        d1 = pltpu.make_async_copy(x_hbm.at[pl.ds(p0, R)], buf.at[pl.ds(0, R)], sem)
        d2 = pltpu.make_async_copy(x_hbm.at[pl.ds(h0, HALO)], buf.at[pl.ds(R, HALO)], sem)
        return d1, d2

    def start_in(c, buf, sem):
        for d in in_copies(c, buf, sem):
            d.start()

    def wait_in(c, buf, sem):
        for d in in_copies(c, buf, sem):
            d.wait()

    def out_copy(c, buf, sem):
        p0 = base + c * R
        return pltpu.make_async_copy(buf, out_hbm.at[pl.ds(p0 * K, R * K)], sem)

    def compute_fast(ib, ob):
        def body_j(j, cc):
            col = pl.ds(j * LANES, LANES)
            for r in range(R):
                a = ib[r, col]
                ob[r * K, col] = a
                for k in range(1, K):
                    b = ib[r + k, col]
                    ob[r * K + k, col] = (a + b) * 0.5
            return cc
        lax.fori_loop(0, NJ, body_j, 0)

    def compute_edge(ib, ob, p0, batch_end):
        def body_j(j, cc):
            col = pl.ds(j * LANES, LANES)
            for r in range(R):
                a = ib[r, col]
                ob[r * K, col] = a
                for k in range(1, K):
                    b = ib[r + k, col]
                    m = jnp.where(p0 + r + k < batch_end,
                                  jnp.float32(0.5), jnp.float32(0.0))
                    ob[r * K + k, col] = a * 0.5 + b * m
            return cc
        lax.fori_loop(0, NJ, body_j, 0)

    start_in(0, in_a, si_a)
    start_in(1, in_b, si_b)

    def pair(cc, carry):
        not_first = cc > 0
        not_last = cc < C // 2 - 1

        c0 = cc * 2
        wait_in(c0, in_a, si_a)
        pl.when(not_first)(lambda: out_copy(c0 - 2, out_a, so_a).wait())
        compute_fast(in_a, out_a)
        out_copy(c0, out_a, so_a).start()
        pl.when(not_last)(lambda: start_in(c0 + 2, in_a, si_a))

        c1 = c0 + 1
        p1 = base + c1 * R
        batch_end = (p1 // L + 1) * L
        is_edge = (p1 + R + K - 2) >= batch_end
        wait_in(c1, in_b, si_b)
        pl.when(not_first)(lambda: out_copy(c1 - 2, out_b, so_b).wait())
        pl.when(jnp.logical_not(is_edge))(lambda: compute_fast(in_b, out_b))
        pl.when(is_edge)(lambda: compute_edge(in_b, out_b, p1, batch_end))
        out_copy(c1, out_b, so_b).start()
        pl.when(not_last)(lambda: start_in(c1 + 2, in_b, si_b))
        return carry

    lax.fori_loop(0, C // 2, pair, 0)

    out_copy(C - 2, out_a, so_a).wait()
    out_copy(C - 1, out_b, so_b).wait()


@jax.jit
def kernel(x):
    xf = x.reshape(P, D)
    mesh = plsc.VectorSubcoreMesh(core_axis_name="c", subcore_axis_name="s")
    run = pl.kernel(
        _body,
        out_type=jax.ShapeDtypeStruct((P * K, D), jnp.float32),
        mesh=mesh,
        scratch_types=[
            pltpu.VMEM((R + HALO, D), jnp.float32),
            pltpu.VMEM((R + HALO, D), jnp.float32),
            pltpu.VMEM((R * K, D), jnp.float32),
            pltpu.VMEM((R * K, D), jnp.float32),
            pltpu.SemaphoreType.DMA,
            pltpu.SemaphoreType.DMA,
            pltpu.SemaphoreType.DMA,
            pltpu.SemaphoreType.DMA,
        ],
    )
    out = run(xf)
    return out.reshape(B, L, K, D)
